# baseline (speedup 1.0000x reference)
import functools

import jax
import jax.numpy as jnp
from jax import lax
from jax.experimental import pallas as pl
from jax.experimental.pallas import tpu as pltpu
from jax.experimental.pallas import tpu_sc as plsc

N = 51200
K = 1000
NC = 2
NS = 16
L = 16
NW = NC * NS
C = N // NW
V1 = C // L
CHUNK = 128
NFULL = C // CHUNK
REM = C - NFULL * CHUNK
ITILES = N // 128


def _build_sc_kernel():
    mesh = plsc.VectorSubcoreMesh(core_axis_name="c", subcore_axis_name="s")

    @functools.partial(
        pl.kernel,
        mesh=mesh,
        out_type=jax.ShapeDtypeStruct((N,), jnp.float32),
        compiler_params=pltpu.CompilerParams(needs_layout_passes=False),
        scratch_types=[
            pltpu.VMEM((C,), jnp.int32),
            pltpu.VMEM((C,), jnp.int32),
            pltpu.VMEM((C,), jnp.float32),
            pltpu.SemaphoreType.DMA,
        ],
    )
    def sc_kernel(prob_hbm, tgt_hbm, out_hbm, tgt_v, idx_v, gat_v, sem):
        wid = lax.axis_index("s") * NC + lax.axis_index("c")
        base = wid * C

        pltpu.sync_copy(tgt_hbm.at[pl.ds(base, C)], tgt_v)

        lane = lax.iota(jnp.int32, L)
        zero_f = jnp.zeros((L,), jnp.float32)

        def idx_body(v, carry):
            for u in range(4):
                off = v * (4 * L) + u * L
                j = tgt_v[pl.ds(off, L)]
                i = (base + off) + lane
                phys = ((((j >> 3) * ITILES + (i >> 7)) << 10)
                        + ((j & 7) << 7) + (i & 127))
                idx_v[pl.ds(off, L)] = phys
            return carry

        lax.fori_loop(0, V1 // 4, idx_body, 0)

        copies = []
        for c in range(NFULL):
            copies.append(pltpu.make_async_copy(
                prob_hbm.at[idx_v.at[pl.ds(c * CHUNK, CHUNK)]],
                gat_v.at[pl.ds(c * CHUNK, CHUNK)], sem))
        if REM:
            copies.append(pltpu.make_async_copy(
                prob_hbm.at[idx_v.at[pl.ds(NFULL * CHUNK, REM)]],
                gat_v.at[pl.ds(NFULL * CHUNK, REM)], sem))
        for cp in copies:
            cp.start()
        for cp in copies:
            cp.wait()

        def mask_body(v, carry):
            for u in range(4):
                off = v * (4 * L) + u * L
                g = gat_v[pl.ds(off, L)]
                t = tgt_v[pl.ds(off, L)]
                gat_v[pl.ds(off, L)] = jnp.where(t == 0, zero_f, g)
            return carry

        lax.fori_loop(0, V1 // 4, mask_body, 0)
        pltpu.sync_copy(gat_v, out_hbm.at[pl.ds(base, C)])

    return sc_kernel


_sc_kernel = _build_sc_kernel()


@jax.jit
def kernel(prob, target, reward):
    tgt = target.astype(jnp.int32)
    rew = reward.reshape((N,))
    prob_lin = (prob.T.reshape(K // 8, 8, ITILES, 128)
                .transpose(0, 2, 1, 3).reshape(N * K))
    gathered = _sc_kernel(prob_lin, tgt)
    return -jnp.sum(gathered * rew)


# speedup vs baseline: 1.0141x; 1.0141x over previous
import functools

import jax
import jax.numpy as jnp
from jax import lax
from jax.experimental import pallas as pl
from jax.experimental.pallas import tpu as pltpu
from jax.experimental.pallas import tpu_sc as plsc

N = 51200
K = 1000
NC = 2
NS = 16
L = 16
NW = NC * NS
C = N // NW
V1 = C // L
CHUNK = 128
NFULL = C // CHUNK
REM = C - NFULL * CHUNK
ITILES = N // 128


def _build_sc_kernel():
    mesh = plsc.VectorSubcoreMesh(core_axis_name="c", subcore_axis_name="s")

    @functools.partial(
        pl.kernel,
        mesh=mesh,
        out_type=jax.ShapeDtypeStruct((N,), jnp.float32),
        compiler_params=pltpu.CompilerParams(needs_layout_passes=False),
        scratch_types=[
            pltpu.VMEM((C,), jnp.int32),
            pltpu.VMEM((C,), jnp.int32),
            pltpu.VMEM((C,), jnp.float32),
            pltpu.SemaphoreType.DMA,
        ],
    )
    def sc_kernel(prob_hbm, tgt_hbm, out_hbm, tgt_v, idx_v, gat_v, sem):
        wid = lax.axis_index("s") * NC + lax.axis_index("c")
        base = wid * C

        pltpu.sync_copy(tgt_hbm.at[pl.ds(base, C)], tgt_v)

        lane = lax.iota(jnp.int32, L)
        zero_f = jnp.zeros((L,), jnp.float32)

        def idx_body(v, carry):
            off = v * L
            j = tgt_v[pl.ds(off, L)]
            i = (base + off) + lane
            phys = ((((j >> 3) * ITILES + (i >> 7)) << 10)
                    + ((j & 7) << 7) + (i & 127))
            idx_v[pl.ds(off, L)] = phys
            return carry

        lax.fori_loop(0, V1, idx_body, 0)

        copies = []
        for c in range(NFULL):
            copies.append(pltpu.make_async_copy(
                prob_hbm.at[idx_v.at[pl.ds(c * CHUNK, CHUNK)]],
                gat_v.at[pl.ds(c * CHUNK, CHUNK)], sem))
        if REM:
            copies.append(pltpu.make_async_copy(
                prob_hbm.at[idx_v.at[pl.ds(NFULL * CHUNK, REM)]],
                gat_v.at[pl.ds(NFULL * CHUNK, REM)], sem))
        for cp in copies:
            cp.start()
        for cp in copies:
            cp.wait()

        def mask_body(v, carry):
            off = v * L
            g = gat_v[pl.ds(off, L)]
            t = tgt_v[pl.ds(off, L)]
            gat_v[pl.ds(off, L)] = jnp.where(t == 0, zero_f, g)
            return carry

        lax.fori_loop(0, V1, mask_body, 0)
        pltpu.sync_copy(gat_v, out_hbm.at[pl.ds(base, C)])

    return sc_kernel


_sc_kernel = _build_sc_kernel()


@jax.jit
def kernel(prob, target, reward):
    tgt = target.astype(jnp.int32)
    rew = reward.reshape((N,))
    prob_lin = (prob.T.reshape(K // 8, 8, ITILES, 128)
                .transpose(0, 2, 1, 3).reshape(N * K))
    gathered = _sc_kernel(prob_lin, tgt)
    return -jnp.sum(gathered * rew)
